# baseline (speedup 1.0000x reference)
import functools

import jax
import jax.numpy as jnp
from jax import lax
from jax.experimental import pallas as pl
from jax.experimental.pallas import tpu as pltpu
from jax.experimental.pallas import tpu_sc as plsc

BATCH = 4096
HIST = 50
EMBED_DIM = 64

_NC, _NS = 2, 16
_NW = _NC * _NS
_BPW = BATCH // _NW
_CB = 16
_NCH = _BPW // _CB


def _sc_gather(idx_hbm, table_hbm, out_hbm, idx_v, buf0, buf1, gsem, osem):
    wid = lax.axis_index("s") * _NC + lax.axis_index("c")
    b0 = wid * _BPW
    pltpu.sync_copy(idx_hbm.at[pl.ds(b0, _BPW)], idx_v)

    bufs = (buf0, buf1)

    def fire(c, buf):
        return [
            pltpu.async_copy(
                table_hbm.at[idx_v.at[c * _CB + k]], buf.at[k], gsem
            )
            for k in range(_CB)
        ]

    gath = {0: fire(0, bufs[0])}
    outs = {}
    for c in range(_NCH):
        b = c & 1
        if c >= 1:
            outs.pop(c - 1).wait()
        if c + 1 < _NCH:
            gath[c + 1] = fire(c + 1, bufs[1 - b])
        for cp in gath.pop(c):
            cp.wait()
        outs[c] = pltpu.async_copy(
            bufs[b], out_hbm.at[pl.ds(b0 + c * _CB, _CB)], osem
        )
    outs.pop(_NCH - 1).wait()


_call = functools.partial(
    pl.kernel,
    mesh=plsc.VectorSubcoreMesh(core_axis_name="c", subcore_axis_name="s"),
    compiler_params=pltpu.CompilerParams(use_tc_tiling_on_sc=False),
    out_type=jax.ShapeDtypeStruct((BATCH, HIST, EMBED_DIM), jnp.float32),
    scratch_types=[
        pltpu.VMEM((_BPW, HIST), jnp.int32),
        pltpu.VMEM((_CB, HIST, EMBED_DIM), jnp.float32),
        pltpu.VMEM((_CB, HIST, EMBED_DIM), jnp.float32),
        pltpu.SemaphoreType.DMA,
        pltpu.SemaphoreType.DMA,
    ],
)(_sc_gather)


def kernel(inputs, table):
    return _call(inputs.astype(jnp.int32), table)
